# baseline (speedup 1.0000x reference)
import numpy as np

import jax
import jax.numpy as jnp
from jax.experimental import pallas as pl
from jax.experimental.pallas import tpu as pltpu

GROUPS = [
    [0, 1], [1, 2, 3, 4, 5], [2, 3, 6], [3, 6, 7], [6, 7], [2, 4, 8],
    [4, 8, 9], [8, 9], [10, 11, 12], [11, 12, 13], [12, 13], [10, 14, 15],
    [14, 15, 16], [15, 16], [5, 10, 11, 14], [2, 5, 10], [0, 1, 2],
]

_B, _J, _C, _D = 16384, 17, 3, 32
_K = _J * _C * _D
_O = 3 * len(GROUPS)

_ROWS, _COLS = [], []
for _i, _g in enumerate(GROUPS):
    for _k, _j in enumerate(_g):
        for _c in range(_C):
            for _d in range(_D):
                _ROWS.append((3 * _j + _c) * _D + _d)
                _COLS.append(3 * _i + _c)
_ROWS = np.asarray(_ROWS, dtype=np.int32)
_COLS = np.asarray(_COLS, dtype=np.int32)

_BT = 512


def _pack_m(weights, biases):
    w_flat = jnp.concatenate([w.reshape(-1) for w in weights])
    m = jnp.zeros((_K, _O), jnp.float32).at[_ROWS, _COLS].add(
        jnp.repeat(w_flat, _D))
    bias_row = jnp.concatenate([jnp.sum(b, axis=0) for b in biases])
    return m, bias_row.reshape(1, _O)


def _tc_body(x_ref, m_ref, b_ref, o_ref):
    o_ref[...] = (
        jnp.dot(x_ref[...], m_ref[...], preferred_element_type=jnp.float32)
        + b_ref[...]
    )


@jax.jit
def _run_tc(x_flat, m, bias_row):
    return pl.pallas_call(
        _tc_body,
        grid=(_B // _BT,),
        in_specs=[
            pl.BlockSpec((_BT, _K), lambda i: (i, 0)),
            pl.BlockSpec((_K, _O), lambda i: (0, 0)),
            pl.BlockSpec((1, _O), lambda i: (0, 0)),
        ],
        out_specs=pl.BlockSpec((_BT, _O), lambda i: (i, 0)),
        out_shape=jax.ShapeDtypeStruct((_B, _O), jnp.float32),
        compiler_params=pltpu.CompilerParams(
            dimension_semantics=("parallel",)),
    )(x_flat, m, bias_row)


def kernel(input, weights, biases):
    m, bias_row = _pack_m(weights, biases)
    x_flat = input.reshape(_B, _K)
    return _run_tc(x_flat, m, bias_row)


# speedup vs baseline: 1.4560x; 1.4560x over previous
import numpy as np

import jax
import jax.numpy as jnp
from jax.experimental import pallas as pl
from jax.experimental.pallas import tpu as pltpu

GROUPS = [
    [0, 1], [1, 2, 3, 4, 5], [2, 3, 6], [3, 6, 7], [6, 7], [2, 4, 8],
    [4, 8, 9], [8, 9], [10, 11, 12], [11, 12, 13], [12, 13], [10, 14, 15],
    [14, 15, 16], [15, 16], [5, 10, 11, 14], [2, 5, 10], [0, 1, 2],
]

_B, _J, _C, _D = 16384, 17, 3, 32
_JC = _J * _C
_K = _JC * _D
_O = 3 * len(GROUPS)

_NW = sum(len(g) for g in GROUPS) * _C
_E_JC = np.zeros((_NW, _JC), dtype=np.float32)
_E_O = np.zeros((_NW, _O), dtype=np.float32)
_m = 0
for _i, _g in enumerate(GROUPS):
    for _j in _g:
        for _c in range(_C):
            _E_JC[_m, 3 * _j + _c] = 1.0
            _E_O[_m, 3 * _i + _c] = 1.0
            _m += 1

_CH = 256
_NCH = _B // _CH
_NBUF = 8


def _pack_m(weights, biases):
    w_flat = jnp.concatenate([w.reshape(-1) for w in weights])
    w51 = jnp.asarray(_E_JC).T @ (w_flat[:, None] * jnp.asarray(_E_O))
    m = jnp.repeat(w51, _D, axis=0)
    bias_row = jnp.concatenate([jnp.sum(b, axis=0) for b in biases])
    return m, bias_row.reshape(1, _O)


def _body(x_hbm, m_ref, b_ref, o_hbm, *scratch):
    ibufs = scratch[0:_NBUF]
    obufs = scratch[_NBUF:2 * _NBUF]
    isems = scratch[2 * _NBUF:3 * _NBUF]
    osems = scratch[3 * _NBUF:4 * _NBUF]

    def in_copy(g, b):
        return pltpu.make_async_copy(
            x_hbm.at[pl.ds(g * _CH, _CH), :], ibufs[b], isems[b])

    def out_copy(g, b):
        return pltpu.make_async_copy(
            obufs[b], o_hbm.at[pl.ds(g * _CH, _CH), :], osems[b])

    for b in range(_NBUF):
        in_copy(b, b).start()

    for g in range(_NCH):
        b = g % _NBUF
        in_copy(g, b).wait()
        if g >= _NBUF:
            out_copy(g - _NBUF, b).wait()
        obufs[b][...] = (
            jnp.dot(ibufs[b][...], m_ref[...],
                    preferred_element_type=jnp.float32)
            + b_ref[...]
        )
        out_copy(g, b).start()
        if g + _NBUF < _NCH:
            in_copy(g + _NBUF, b).start()

    for g in range(_NCH - _NBUF, _NCH):
        out_copy(g, g % _NBUF).wait()


@jax.jit
def _run_tc(x_flat, m, bias_row):
    return pl.pallas_call(
        _body,
        in_specs=[
            pl.BlockSpec(memory_space=pl.ANY),
            pl.BlockSpec(memory_space=pltpu.VMEM),
            pl.BlockSpec(memory_space=pltpu.VMEM),
        ],
        out_specs=pl.BlockSpec(memory_space=pl.ANY),
        out_shape=jax.ShapeDtypeStruct((_B, _O), jnp.float32),
        scratch_shapes=(
            [pltpu.VMEM((_CH, _K), jnp.float32) for _ in range(_NBUF)]
            + [pltpu.VMEM((_CH, _O), jnp.float32) for _ in range(_NBUF)]
            + [pltpu.SemaphoreType.DMA for _ in range(2 * _NBUF)]
        ),
    )(x_flat, m, bias_row)


def kernel(input, weights, biases):
    m, bias_row = _pack_m(weights, biases)
    x_flat = input.reshape(_B, _K)
    return _run_tc(x_flat, m, bias_row)
